# Initial kernel scaffold; baseline (speedup 1.0000x reference)
import jax
import jax.numpy as jnp
from jax.experimental import pallas as pl


def kernel(x, hg, W1, b1, W2, b2):
    raise NotImplementedError("write your pallas kernel here")


# speedup vs baseline: 4.6830x; 4.6830x over previous
import jax
import jax.numpy as jnp
from jax import lax
from jax.experimental import pallas as pl
from jax.experimental.pallas import tpu as pltpu
from jax.experimental.pallas import tpu_sc as plsc

N = 10000
M = 10000
E = 320000
D = 128

CHUNK = 80
NCHUNKS = E // CHUNK
NC, NS = 2, 16
NW = NC * NS
WCHUNKS = NCHUNKS // NW
SUB_SPAN = 624
TAIL = M - SUB_SPAN * NS
EPW = E // NW
HR = 80

_sc_mesh = plsc.VectorSubcoreMesh(core_axis_name="c", subcore_axis_name="s")


def _make_seg_sum():
  scratch = [
      pltpu.VMEM((CHUNK,), jnp.int32),
      pltpu.VMEM((CHUNK,), jnp.int32),
      pltpu.VMEM((CHUNK, D), jnp.float32),
      pltpu.VMEM_SHARED((M, D), jnp.float32),
      pltpu.SemaphoreType.DMA,
  ]

  def body(src_hbm, gidx_hbm, sidx_hbm, zd_hbm, out_hbm,
           gbuf, sbuf, rows, acc, sem):
    cid = lax.axis_index("c")
    sid = lax.axis_index("s")
    wid = cid * NS + sid

    acc_base = sid * SUB_SPAN
    pltpu.sync_copy(zd_hbm.at[pl.ds(0, SUB_SPAN)],
                    acc.at[pl.ds(acc_base, SUB_SPAN)])

    @pl.when(sid == NS - 1)
    def _():
      pltpu.sync_copy(zd_hbm.at[pl.ds(0, TAIL)],
                      acc.at[pl.ds(NS * SUB_SPAN, TAIL)])

    plsc.subcore_barrier()

    off_w = wid * WCHUNKS

    @pl.loop(0, WCHUNKS)
    def _(j):
      base = (off_w + j) * CHUNK
      pltpu.sync_copy(gidx_hbm.at[pl.ds(base, CHUNK)], gbuf)
      pltpu.sync_copy(sidx_hbm.at[pl.ds(base, CHUNK)], sbuf)
      pltpu.async_copy(src_hbm.at[gbuf], rows, sem).wait()
      pltpu.sync_copy(rows, acc.at[sbuf], add=True)

    plsc.subcore_barrier()

    out_base = cid * M + acc_base
    pltpu.sync_copy(acc.at[pl.ds(acc_base, SUB_SPAN)],
                    out_hbm.at[pl.ds(out_base, SUB_SPAN)])

    @pl.when(sid == NS - 1)
    def _():
      pltpu.sync_copy(acc.at[pl.ds(NS * SUB_SPAN, TAIL)],
                      out_hbm.at[pl.ds(cid * M + NS * SUB_SPAN, TAIL)])

  return pl.kernel(body, out_type=jax.ShapeDtypeStruct((NC * M, D), jnp.float32),
                   mesh=_sc_mesh, scratch_types=scratch, name="seg_sum")


def _make_counts():
  scratch = [
      pltpu.VMEM((EPW,), jnp.int32),
      pltpu.VMEM((HR, 128), jnp.float32),
  ]

  def body(sidx_hbm, zd_hbm, out_hbm, sbuf, hist):
    cid = lax.axis_index("c")
    sid = lax.axis_index("s")
    wid = cid * NS + sid

    pltpu.sync_copy(zd_hbm.at[pl.ds(0, HR)], hist)
    pltpu.sync_copy(sidx_hbm.at[pl.ds(wid * EPW, EPW)], sbuf)

    ones = jnp.ones((16,), jnp.float32)

    @pl.loop(0, EPW // 16)
    def _(i):
      idx = sbuf[pl.ds(i * 16, 16)]
      hi = lax.shift_right_logical(idx, 7)
      lo = lax.bitwise_and(idx, 127)
      plsc.addupdate_scatter(hist, [hi, lo], ones)

    pltpu.sync_copy(hist, out_hbm.at[pl.ds(wid * HR, HR)])

  return pl.kernel(
      body, out_type=jax.ShapeDtypeStruct((NW * HR, 128), jnp.float32),
      mesh=_sc_mesh, scratch_types=scratch, name="edge_counts",
      compiler_params=pltpu.CompilerParams(needs_layout_passes=False))


_seg_sum = _make_seg_sum()
_counts = _make_counts()


_BN = 1000
_G = N // _BN


def _mm_body(x_ref, w_ref, b_ref, o_ref):
  o_ref[...] = jnp.dot(x_ref[...], w_ref[...],
                       preferred_element_type=jnp.float32) + b_ref[...]


def _matmul(x, W, b):
  return pl.pallas_call(
      _mm_body,
      grid=(_G,),
      in_specs=[
          pl.BlockSpec((_BN, D), lambda i: (i, 0)),
          pl.BlockSpec((D, D), lambda i: (0, 0)),
          pl.BlockSpec((1, D), lambda i: (0, 0)),
      ],
      out_specs=pl.BlockSpec((_BN, D), lambda i: (i, 0)),
      out_shape=jax.ShapeDtypeStruct((N, D), jnp.float32),
  )(x, W, b.reshape(1, D))


def _div_body(p_ref, q_ref, c_ref, o_ref):
  o_ref[...] = (p_ref[0] + q_ref[0]) / jnp.maximum(c_ref[...], 1.0)


def _combine_div(parts, cnt):
  return pl.pallas_call(
      _div_body,
      grid=(_G,),
      in_specs=[
          pl.BlockSpec((1, _BN, D), lambda i: (0, i, 0)),
          pl.BlockSpec((1, _BN, D), lambda i: (1, i, 0)),
          pl.BlockSpec((_BN, 1), lambda i: (i, 0)),
      ],
      out_specs=pl.BlockSpec((_BN, D), lambda i: (i, 0)),
      out_shape=jax.ShapeDtypeStruct((M, D), jnp.float32),
  )(parts, parts, cnt)


def _resmm_body(h_ref, p_ref, q_ref, w_ref, b_ref, o_ref):
  a = jnp.maximum(h_ref[...] + p_ref[0] + q_ref[0], 0.0)
  o_ref[...] = jnp.dot(a, w_ref[...],
                       preferred_element_type=jnp.float32) + b_ref[...]


def _residual_relu_matmul(h, parts, W, b):
  return pl.pallas_call(
      _resmm_body,
      grid=(_G,),
      in_specs=[
          pl.BlockSpec((_BN, D), lambda i: (i, 0)),
          pl.BlockSpec((1, _BN, D), lambda i: (0, i, 0)),
          pl.BlockSpec((1, _BN, D), lambda i: (1, i, 0)),
          pl.BlockSpec((D, D), lambda i: (0, 0)),
          pl.BlockSpec((1, D), lambda i: (0, 0)),
      ],
      out_specs=pl.BlockSpec((_BN, D), lambda i: (i, 0)),
      out_shape=jax.ShapeDtypeStruct((N, D), jnp.float32),
  )(h, parts, parts, W, b.reshape(1, D))


def _resrelu_body(h_ref, p_ref, q_ref, o_ref):
  o_ref[...] = jnp.maximum(h_ref[...] + p_ref[0] + q_ref[0], 0.0)


def _residual_relu(h, parts):
  return pl.pallas_call(
      _resrelu_body,
      grid=(_G,),
      in_specs=[
          pl.BlockSpec((_BN, D), lambda i: (i, 0)),
          pl.BlockSpec((1, _BN, D), lambda i: (0, i, 0)),
          pl.BlockSpec((1, _BN, D), lambda i: (1, i, 0)),
      ],
      out_specs=pl.BlockSpec((_BN, D), lambda i: (i, 0)),
      out_shape=jax.ShapeDtypeStruct((N, D), jnp.float32),
  )(h, parts, parts)


@jax.jit
def kernel(x, hg, W1, b1, W2, b2):
  v1d = hg[0]
  e1d = hg[1]
  zd = jnp.zeros((SUB_SPAN + TAIL, D), jnp.float32)

  hist = _counts(e1d, zd)
  cnt = hist.reshape(NW, HR * 128).sum(axis=0)[:M].reshape(M, 1)

  h1 = _matmul(x, W1, b1)
  ep = _seg_sum(h1, v1d, e1d, zd)
  y1 = _combine_div(ep.reshape(NC, M, D), cnt)
  vp = _seg_sum(y1, e1d, v1d, zd)
  h2 = _residual_relu_matmul(h1, vp.reshape(NC, N, D), W2, b2)

  ep2 = _seg_sum(h2, v1d, e1d, zd)
  y2 = _combine_div(ep2.reshape(NC, M, D), cnt)
  vp2 = _seg_sum(y2, e1d, v1d, zd)
  return _residual_relu(h2, vp2.reshape(NC, N, D))


# speedup vs baseline: 6.9925x; 1.4932x over previous
import jax
import jax.numpy as jnp
from jax import lax
from jax.experimental import pallas as pl
from jax.experimental.pallas import tpu as pltpu
from jax.experimental.pallas import tpu_sc as plsc

N = 10000
M = 10000
E = 320000
D = 128

CHUNK = 200
NCHUNKS = E // CHUNK
NC, NS = 2, 16
NW = NC * NS
WCHUNKS = NCHUNKS // NW
SUB_SPAN = 624
TAIL = M - SUB_SPAN * NS
EPW = E // NW
HR = 80

_sc_mesh = plsc.VectorSubcoreMesh(core_axis_name="c", subcore_axis_name="s")


def _make_seg_sum():
  scratch = [
      pltpu.VMEM((CHUNK,), jnp.int32),
      pltpu.VMEM((CHUNK,), jnp.int32),
      pltpu.VMEM((CHUNK, D), jnp.float32),
      pltpu.VMEM_SHARED((M, D), jnp.float32),
      pltpu.SemaphoreType.DMA,
  ]

  def body(src_hbm, gidx_hbm, sidx_hbm, zd_hbm, out_hbm,
           gbuf, sbuf, rows, acc, sem):
    cid = lax.axis_index("c")
    sid = lax.axis_index("s")
    wid = cid * NS + sid

    acc_base = sid * SUB_SPAN
    pltpu.sync_copy(zd_hbm.at[pl.ds(0, SUB_SPAN)],
                    acc.at[pl.ds(acc_base, SUB_SPAN)])

    @pl.when(sid == NS - 1)
    def _():
      pltpu.sync_copy(zd_hbm.at[pl.ds(0, TAIL)],
                      acc.at[pl.ds(NS * SUB_SPAN, TAIL)])

    plsc.subcore_barrier()

    off_w = wid * WCHUNKS

    @pl.loop(0, WCHUNKS)
    def _(j):
      base = (off_w + j) * CHUNK
      pltpu.sync_copy(gidx_hbm.at[pl.ds(base, CHUNK)], gbuf)
      pltpu.sync_copy(sidx_hbm.at[pl.ds(base, CHUNK)], sbuf)
      pltpu.async_copy(src_hbm.at[gbuf], rows, sem).wait()
      pltpu.sync_copy(rows, acc.at[sbuf], add=True)

    plsc.subcore_barrier()

    out_base = cid * M + acc_base
    pltpu.sync_copy(acc.at[pl.ds(acc_base, SUB_SPAN)],
                    out_hbm.at[pl.ds(out_base, SUB_SPAN)])

    @pl.when(sid == NS - 1)
    def _():
      pltpu.sync_copy(acc.at[pl.ds(NS * SUB_SPAN, TAIL)],
                      out_hbm.at[pl.ds(cid * M + NS * SUB_SPAN, TAIL)])

  return pl.kernel(body, out_type=jax.ShapeDtypeStruct((NC * M, D), jnp.float32),
                   mesh=_sc_mesh, scratch_types=scratch, name="seg_sum")


def _make_counts():
  scratch = [
      pltpu.VMEM((EPW,), jnp.int32),
      pltpu.VMEM((HR, 128), jnp.float32),
  ]

  def body(sidx_hbm, zd_hbm, out_hbm, sbuf, hist):
    cid = lax.axis_index("c")
    sid = lax.axis_index("s")
    wid = cid * NS + sid

    pltpu.sync_copy(zd_hbm.at[pl.ds(0, HR)], hist)
    pltpu.sync_copy(sidx_hbm.at[pl.ds(wid * EPW, EPW)], sbuf)

    ones = jnp.ones((16,), jnp.float32)

    @pl.loop(0, EPW // 16)
    def _(i):
      idx = sbuf[pl.ds(i * 16, 16)]
      hi = lax.shift_right_logical(idx, 7)
      lo = lax.bitwise_and(idx, 127)
      plsc.addupdate_scatter(hist, [hi, lo], ones)

    pltpu.sync_copy(hist, out_hbm.at[pl.ds(wid * HR, HR)])

  return pl.kernel(
      body, out_type=jax.ShapeDtypeStruct((NW * HR, 128), jnp.float32),
      mesh=_sc_mesh, scratch_types=scratch, name="edge_counts",
      compiler_params=pltpu.CompilerParams(needs_layout_passes=False))


_seg_sum = _make_seg_sum()
_counts = _make_counts()


_BN = 1000
_G = N // _BN


def _mm_body(x_ref, w_ref, b_ref, o_ref):
  o_ref[...] = jnp.dot(x_ref[...], w_ref[...],
                       preferred_element_type=jnp.float32) + b_ref[...]


def _matmul(x, W, b):
  return pl.pallas_call(
      _mm_body,
      grid=(_G,),
      in_specs=[
          pl.BlockSpec((_BN, D), lambda i: (i, 0)),
          pl.BlockSpec((D, D), lambda i: (0, 0)),
          pl.BlockSpec((1, D), lambda i: (0, 0)),
      ],
      out_specs=pl.BlockSpec((_BN, D), lambda i: (i, 0)),
      out_shape=jax.ShapeDtypeStruct((N, D), jnp.float32),
  )(x, W, b.reshape(1, D))


def _div_body(p_ref, q_ref, c_ref, o_ref):
  o_ref[...] = (p_ref[0] + q_ref[0]) / jnp.maximum(c_ref[...], 1.0)


def _combine_div(parts, cnt):
  return pl.pallas_call(
      _div_body,
      grid=(_G,),
      in_specs=[
          pl.BlockSpec((1, _BN, D), lambda i: (0, i, 0)),
          pl.BlockSpec((1, _BN, D), lambda i: (1, i, 0)),
          pl.BlockSpec((_BN, 1), lambda i: (i, 0)),
      ],
      out_specs=pl.BlockSpec((_BN, D), lambda i: (i, 0)),
      out_shape=jax.ShapeDtypeStruct((M, D), jnp.float32),
  )(parts, parts, cnt)


def _resmm_body(h_ref, p_ref, q_ref, w_ref, b_ref, o_ref):
  a = jnp.maximum(h_ref[...] + p_ref[0] + q_ref[0], 0.0)
  o_ref[...] = jnp.dot(a, w_ref[...],
                       preferred_element_type=jnp.float32) + b_ref[...]


def _residual_relu_matmul(h, parts, W, b):
  return pl.pallas_call(
      _resmm_body,
      grid=(_G,),
      in_specs=[
          pl.BlockSpec((_BN, D), lambda i: (i, 0)),
          pl.BlockSpec((1, _BN, D), lambda i: (0, i, 0)),
          pl.BlockSpec((1, _BN, D), lambda i: (1, i, 0)),
          pl.BlockSpec((D, D), lambda i: (0, 0)),
          pl.BlockSpec((1, D), lambda i: (0, 0)),
      ],
      out_specs=pl.BlockSpec((_BN, D), lambda i: (i, 0)),
      out_shape=jax.ShapeDtypeStruct((N, D), jnp.float32),
  )(h, parts, parts, W, b.reshape(1, D))


def _resrelu_body(h_ref, p_ref, q_ref, o_ref):
  o_ref[...] = jnp.maximum(h_ref[...] + p_ref[0] + q_ref[0], 0.0)


def _residual_relu(h, parts):
  return pl.pallas_call(
      _resrelu_body,
      grid=(_G,),
      in_specs=[
          pl.BlockSpec((_BN, D), lambda i: (i, 0)),
          pl.BlockSpec((1, _BN, D), lambda i: (0, i, 0)),
          pl.BlockSpec((1, _BN, D), lambda i: (1, i, 0)),
      ],
      out_specs=pl.BlockSpec((_BN, D), lambda i: (i, 0)),
      out_shape=jax.ShapeDtypeStruct((N, D), jnp.float32),
  )(h, parts, parts)


@jax.jit
def kernel(x, hg, W1, b1, W2, b2):
  v1d = hg[0]
  e1d = hg[1]
  zd = jnp.zeros((SUB_SPAN + TAIL, D), jnp.float32)

  hist = _counts(e1d, zd)
  cnt = hist.reshape(NW, HR * 128).sum(axis=0)[:M].reshape(M, 1)

  h1 = _matmul(x, W1, b1)
  ep = _seg_sum(h1, v1d, e1d, zd)
  y1 = _combine_div(ep.reshape(NC, M, D), cnt)
  vp = _seg_sum(y1, e1d, v1d, zd)
  h2 = _residual_relu_matmul(h1, vp.reshape(NC, N, D), W2, b2)

  ep2 = _seg_sum(h2, v1d, e1d, zd)
  y2 = _combine_div(ep2.reshape(NC, M, D), cnt)
  vp2 = _seg_sum(y2, e1d, v1d, zd)
  return _residual_relu(h2, vp2.reshape(NC, N, D))


# speedup vs baseline: 7.8368x; 1.1207x over previous
import jax
import jax.numpy as jnp
from jax import lax
from jax.experimental import pallas as pl
from jax.experimental.pallas import tpu as pltpu
from jax.experimental.pallas import tpu_sc as plsc

N = 10000
M = 10000
E = 320000
D = 128

CHUNK = 160
NCHUNKS = E // CHUNK
NC, NS = 2, 16
NW = NC * NS
WCHUNKS = NCHUNKS // NW
LEFT = NCHUNKS - WCHUNKS * NW
SUB_SPAN = 624
TAIL = M - SUB_SPAN * NS
EPW = E // NW
HR = 80

_sc_mesh = plsc.VectorSubcoreMesh(core_axis_name="c", subcore_axis_name="s")


def _make_seg_sum():
  scratch = [
      pltpu.VMEM((CHUNK,), jnp.int32),
      pltpu.VMEM((CHUNK,), jnp.int32),
      pltpu.VMEM((CHUNK,), jnp.int32),
      pltpu.VMEM((CHUNK,), jnp.int32),
      pltpu.VMEM((CHUNK, D), jnp.float32),
      pltpu.VMEM((CHUNK, D), jnp.float32),
      pltpu.VMEM_SHARED((M, D), jnp.float32),
      pltpu.SemaphoreType.DMA,
      pltpu.SemaphoreType.DMA,
      pltpu.SemaphoreType.DMA,
      pltpu.SemaphoreType.DMA,
  ]

  def body(src_hbm, gidx_hbm, sidx_hbm, zd_hbm, out_hbm,
           gb0, sb0, gb1, sb1, r0, r1, acc, mg0, mg1, ms0, ms1):
    cid = lax.axis_index("c")
    sid = lax.axis_index("s")
    wid = cid * NS + sid

    gb = (gb0, gb1)
    sb = (sb0, sb1)
    rows = (r0, r1)
    mg = (mg0, mg1)
    ms = (ms0, ms1)

    acc_base = sid * SUB_SPAN
    pltpu.sync_copy(zd_hbm.at[pl.ds(0, SUB_SPAN)],
                    acc.at[pl.ds(acc_base, SUB_SPAN)])

    @pl.when(sid == NS - 1)
    def _():
      pltpu.sync_copy(zd_hbm.at[pl.ds(0, TAIL)],
                      acc.at[pl.ds(NS * SUB_SPAN, TAIL)])

    plsc.subcore_barrier()

    off_w = wid * WCHUNKS

    def load_idx(j, b):
      base = (off_w + j) * CHUNK
      pltpu.sync_copy(gidx_hbm.at[pl.ds(base, CHUNK)], gb[b])
      pltpu.sync_copy(sidx_hbm.at[pl.ds(base, CHUNK)], sb[b])

    def gather(b):
      pltpu.async_copy(src_hbm.at[gb[b]], rows[b], mg[b])

    def gather_wait(b):
      pltpu.make_async_copy(src_hbm.at[gb[b]], rows[b], mg[b]).wait()

    def scatter(b):
      pltpu.async_copy(rows[b], acc.at[sb[b]], ms[b], add=True)

    def scatter_wait(b):
      pltpu.make_async_copy(rows[b], acc.at[sb[b]], ms[b]).wait()

    load_idx(0, 0)
    gather(0)

    @pl.loop(0, WCHUNKS // 2)
    def _(k):
      gather_wait(0)

      @pl.when(k > 0)
      def _():
        scatter_wait(1)

      load_idx(2 * k + 1, 1)
      gather(1)
      scatter(0)

      gather_wait(1)
      scatter_wait(0)

      @pl.when(k < WCHUNKS // 2 - 1)
      def _():
        load_idx(2 * k + 2, 0)
        gather(0)

      scatter(1)

    scatter_wait(1)

    @pl.when(wid < LEFT)
    def _():
      base = (NW * WCHUNKS + wid) * CHUNK
      pltpu.sync_copy(gidx_hbm.at[pl.ds(base, CHUNK)], gb0)
      pltpu.sync_copy(sidx_hbm.at[pl.ds(base, CHUNK)], sb0)
      pltpu.async_copy(src_hbm.at[gb0], r0, mg0).wait()
      pltpu.sync_copy(r0, acc.at[sb0], add=True)

    plsc.subcore_barrier()

    out_base = cid * M + acc_base
    pltpu.sync_copy(acc.at[pl.ds(acc_base, SUB_SPAN)],
                    out_hbm.at[pl.ds(out_base, SUB_SPAN)])

    @pl.when(sid == NS - 1)
    def _():
      pltpu.sync_copy(acc.at[pl.ds(NS * SUB_SPAN, TAIL)],
                      out_hbm.at[pl.ds(cid * M + NS * SUB_SPAN, TAIL)])

  return pl.kernel(body, out_type=jax.ShapeDtypeStruct((NC * M, D), jnp.float32),
                   mesh=_sc_mesh, scratch_types=scratch, name="seg_sum")


def _make_counts():
  scratch = [
      pltpu.VMEM((EPW,), jnp.int32),
      pltpu.VMEM((HR, 128), jnp.float32),
  ]

  def body(sidx_hbm, zd_hbm, out_hbm, sbuf, hist):
    cid = lax.axis_index("c")
    sid = lax.axis_index("s")
    wid = cid * NS + sid

    pltpu.sync_copy(zd_hbm.at[pl.ds(0, HR)], hist)
    pltpu.sync_copy(sidx_hbm.at[pl.ds(wid * EPW, EPW)], sbuf)

    ones = jnp.ones((16,), jnp.float32)

    @pl.loop(0, EPW // 16)
    def _(i):
      idx = sbuf[pl.ds(i * 16, 16)]
      hi = lax.shift_right_logical(idx, 7)
      lo = lax.bitwise_and(idx, 127)
      plsc.addupdate_scatter(hist, [hi, lo], ones)

    pltpu.sync_copy(hist, out_hbm.at[pl.ds(wid * HR, HR)])

  return pl.kernel(
      body, out_type=jax.ShapeDtypeStruct((NW * HR, 128), jnp.float32),
      mesh=_sc_mesh, scratch_types=scratch, name="edge_counts",
      compiler_params=pltpu.CompilerParams(needs_layout_passes=False))


_seg_sum = _make_seg_sum()
_counts = _make_counts()


_BN = 1000
_G = N // _BN


def _mm_body(x_ref, w_ref, b_ref, o_ref):
  o_ref[...] = jnp.dot(x_ref[...], w_ref[...],
                       preferred_element_type=jnp.float32) + b_ref[...]


def _matmul(x, W, b):
  return pl.pallas_call(
      _mm_body,
      grid=(_G,),
      in_specs=[
          pl.BlockSpec((_BN, D), lambda i: (i, 0)),
          pl.BlockSpec((D, D), lambda i: (0, 0)),
          pl.BlockSpec((1, D), lambda i: (0, 0)),
      ],
      out_specs=pl.BlockSpec((_BN, D), lambda i: (i, 0)),
      out_shape=jax.ShapeDtypeStruct((N, D), jnp.float32),
  )(x, W, b.reshape(1, D))


def _div_body(p_ref, q_ref, c_ref, o_ref):
  o_ref[...] = (p_ref[0] + q_ref[0]) / jnp.maximum(c_ref[...], 1.0)


def _combine_div(parts, cnt):
  return pl.pallas_call(
      _div_body,
      grid=(_G,),
      in_specs=[
          pl.BlockSpec((1, _BN, D), lambda i: (0, i, 0)),
          pl.BlockSpec((1, _BN, D), lambda i: (1, i, 0)),
          pl.BlockSpec((_BN, 1), lambda i: (i, 0)),
      ],
      out_specs=pl.BlockSpec((_BN, D), lambda i: (i, 0)),
      out_shape=jax.ShapeDtypeStruct((M, D), jnp.float32),
  )(parts, parts, cnt)


def _resmm_body(h_ref, p_ref, q_ref, w_ref, b_ref, o_ref):
  a = jnp.maximum(h_ref[...] + p_ref[0] + q_ref[0], 0.0)
  o_ref[...] = jnp.dot(a, w_ref[...],
                       preferred_element_type=jnp.float32) + b_ref[...]


def _residual_relu_matmul(h, parts, W, b):
  return pl.pallas_call(
      _resmm_body,
      grid=(_G,),
      in_specs=[
          pl.BlockSpec((_BN, D), lambda i: (i, 0)),
          pl.BlockSpec((1, _BN, D), lambda i: (0, i, 0)),
          pl.BlockSpec((1, _BN, D), lambda i: (1, i, 0)),
          pl.BlockSpec((D, D), lambda i: (0, 0)),
          pl.BlockSpec((1, D), lambda i: (0, 0)),
      ],
      out_specs=pl.BlockSpec((_BN, D), lambda i: (i, 0)),
      out_shape=jax.ShapeDtypeStruct((N, D), jnp.float32),
  )(h, parts, parts, W, b.reshape(1, D))


def _resrelu_body(h_ref, p_ref, q_ref, o_ref):
  o_ref[...] = jnp.maximum(h_ref[...] + p_ref[0] + q_ref[0], 0.0)


def _residual_relu(h, parts):
  return pl.pallas_call(
      _resrelu_body,
      grid=(_G,),
      in_specs=[
          pl.BlockSpec((_BN, D), lambda i: (i, 0)),
          pl.BlockSpec((1, _BN, D), lambda i: (0, i, 0)),
          pl.BlockSpec((1, _BN, D), lambda i: (1, i, 0)),
      ],
      out_specs=pl.BlockSpec((_BN, D), lambda i: (i, 0)),
      out_shape=jax.ShapeDtypeStruct((N, D), jnp.float32),
  )(h, parts, parts)


@jax.jit
def kernel(x, hg, W1, b1, W2, b2):
  v1d = hg[0]
  e1d = hg[1]
  zd = jnp.zeros((SUB_SPAN + TAIL, D), jnp.float32)

  hist = _counts(e1d, zd)
  cnt = hist.reshape(NW, HR * 128).sum(axis=0)[:M].reshape(M, 1)

  h1 = _matmul(x, W1, b1)
  ep = _seg_sum(h1, v1d, e1d, zd)
  y1 = _combine_div(ep.reshape(NC, M, D), cnt)
  vp = _seg_sum(y1, e1d, v1d, zd)
  h2 = _residual_relu_matmul(h1, vp.reshape(NC, N, D), W2, b2)

  ep2 = _seg_sum(h2, v1d, e1d, zd)
  y2 = _combine_div(ep2.reshape(NC, M, D), cnt)
  vp2 = _seg_sum(y2, e1d, v1d, zd)
  return _residual_relu(h2, vp2.reshape(NC, N, D))


# speedup vs baseline: 10.6652x; 1.3609x over previous
import jax
import jax.numpy as jnp
from jax import lax
from jax.experimental import pallas as pl
from jax.experimental.pallas import tpu as pltpu
from jax.experimental.pallas import tpu_sc as plsc

N = 10000
M = 10000
E = 320000
D = 128

CHUNK = 160
NCHUNKS = E // CHUNK
NC, NS = 2, 16
NW = NC * NS
WCHUNKS = NCHUNKS // NW
LEFT = NCHUNKS - WCHUNKS * NW
SUB_SPAN = 624
TAIL = M - SUB_SPAN * NS
EPW = E // NW
HR = 80

_sc_mesh = plsc.VectorSubcoreMesh(core_axis_name="c", subcore_axis_name="s")


def _make_seg_sum():
  scratch = (
      [pltpu.VMEM((CHUNK,), jnp.int32) for _ in range(8)] +
      [pltpu.VMEM((CHUNK, D), jnp.float32) for _ in range(2)] +
      [pltpu.VMEM_SHARED((M, D), jnp.float32)] +
      [pltpu.SemaphoreType.DMA for _ in range(8)]
  )

  def body(src_hbm, gidx_hbm, sidx_hbm, zd_hbm, out_hbm, *refs):
    gb = refs[0:4]
    sb = refs[4:8]
    rows = refs[8:10]
    acc = refs[10]
    mg = refs[11:13]
    ms = refs[13:15]
    mi = refs[15:19]

    cid = lax.axis_index("c")
    sid = lax.axis_index("s")
    wid = cid * NS + sid

    acc_base = sid * SUB_SPAN
    pltpu.sync_copy(zd_hbm.at[pl.ds(0, SUB_SPAN)],
                    acc.at[pl.ds(acc_base, SUB_SPAN)])

    @pl.when(sid == NS - 1)
    def _():
      pltpu.sync_copy(zd_hbm.at[pl.ds(0, TAIL)],
                      acc.at[pl.ds(NS * SUB_SPAN, TAIL)])

    plsc.subcore_barrier()

    off_w = wid * WCHUNKS

    def idx_start(j, p):
      base = (off_w + j) * CHUNK
      pltpu.async_copy(gidx_hbm.at[pl.ds(base, CHUNK)], gb[p], mi[p])
      pltpu.async_copy(sidx_hbm.at[pl.ds(base, CHUNK)], sb[p], mi[p])

    def idx_wait(j, p):
      base = (off_w + j) * CHUNK
      pltpu.make_async_copy(gidx_hbm.at[pl.ds(base, CHUNK)], gb[p],
                            mi[p]).wait()
      pltpu.make_async_copy(sidx_hbm.at[pl.ds(base, CHUNK)], sb[p],
                            mi[p]).wait()

    def gather(p, b):
      pltpu.async_copy(src_hbm.at[gb[p]], rows[b], mg[b])

    def gather_wait(p, b):
      pltpu.make_async_copy(src_hbm.at[gb[p]], rows[b], mg[b]).wait()

    def scatter(p, b):
      pltpu.async_copy(rows[b], acc.at[sb[p]], ms[b], add=True)

    def scatter_wait(p, b):
      pltpu.make_async_copy(rows[b], acc.at[sb[p]], ms[b]).wait()

    idx_start(0, 0)
    idx_start(1, 1)
    idx_start(2, 2)
    idx_wait(0, 0)
    gather(0, 0)

    @pl.loop(0, WCHUNKS // 4)
    def _(k):
      for t in range(4):
        j = 4 * k + t
        b = t % 2
        gather_wait(t, b)
        if t == 0:
          @pl.when(k > 0)
          def _():
            scatter_wait(3, 1 - b)
        else:
          scatter_wait(t - 1, 1 - b)
        if t == 3:
          @pl.when(k < WCHUNKS // 4 - 1)
          def _():
            idx_start(j + 3, (t + 3) % 4)
        else:
          idx_start(j + 3, (t + 3) % 4)
        idx_wait(j + 1, (t + 1) % 4)
        gather((t + 1) % 4, 1 - b)
        scatter(t, b)

    j = WCHUNKS - 2
    gather_wait(0, 0)
    scatter_wait(3, 1)
    idx_wait(j + 1, 1)
    gather(1, 1)
    scatter(0, 0)
    gather_wait(1, 1)
    scatter_wait(0, 0)
    scatter(1, 1)
    scatter_wait(1, 1)

    @pl.when(wid < LEFT)
    def _():
      base = (NW * WCHUNKS + wid) * CHUNK
      pltpu.sync_copy(gidx_hbm.at[pl.ds(base, CHUNK)], gb[0])
      pltpu.sync_copy(sidx_hbm.at[pl.ds(base, CHUNK)], sb[0])
      pltpu.async_copy(src_hbm.at[gb[0]], rows[0], mg[0]).wait()
      pltpu.sync_copy(rows[0], acc.at[sb[0]], add=True)

    plsc.subcore_barrier()

    out_base = cid * M + acc_base
    pltpu.sync_copy(acc.at[pl.ds(acc_base, SUB_SPAN)],
                    out_hbm.at[pl.ds(out_base, SUB_SPAN)])

    @pl.when(sid == NS - 1)
    def _():
      pltpu.sync_copy(acc.at[pl.ds(NS * SUB_SPAN, TAIL)],
                      out_hbm.at[pl.ds(cid * M + NS * SUB_SPAN, TAIL)])

  return pl.kernel(body, out_type=jax.ShapeDtypeStruct((NC * M, D), jnp.float32),
                   mesh=_sc_mesh, scratch_types=scratch, name="seg_sum")


def _make_counts():
  scratch = [
      pltpu.VMEM((EPW,), jnp.int32),
      pltpu.VMEM((HR, 128), jnp.float32),
  ]

  def body(sidx_hbm, zd_hbm, out_hbm, sbuf, hist):
    cid = lax.axis_index("c")
    sid = lax.axis_index("s")
    wid = cid * NS + sid

    pltpu.sync_copy(zd_hbm.at[pl.ds(0, HR)], hist)
    pltpu.sync_copy(sidx_hbm.at[pl.ds(wid * EPW, EPW)], sbuf)

    ones = jnp.ones((16,), jnp.float32)

    @pl.loop(0, EPW // 16)
    def _(i):
      idx = sbuf[pl.ds(i * 16, 16)]
      hi = lax.shift_right_logical(idx, 7)
      lo = lax.bitwise_and(idx, 127)
      plsc.addupdate_scatter(hist, [hi, lo], ones)

    pltpu.sync_copy(hist, out_hbm.at[pl.ds(wid * HR, HR)])

  return pl.kernel(
      body, out_type=jax.ShapeDtypeStruct((NW * HR, 128), jnp.float32),
      mesh=_sc_mesh, scratch_types=scratch, name="edge_counts",
      compiler_params=pltpu.CompilerParams(needs_layout_passes=False))


_seg_sum = _make_seg_sum()
_counts = _make_counts()


_BN = 1000
_G = N // _BN


def _mm_body(x_ref, w_ref, b_ref, o_ref):
  o_ref[...] = jnp.dot(x_ref[...], w_ref[...],
                       preferred_element_type=jnp.float32) + b_ref[...]


def _matmul(x, W, b):
  return pl.pallas_call(
      _mm_body,
      grid=(_G,),
      in_specs=[
          pl.BlockSpec((_BN, D), lambda i: (i, 0)),
          pl.BlockSpec((D, D), lambda i: (0, 0)),
          pl.BlockSpec((1, D), lambda i: (0, 0)),
      ],
      out_specs=pl.BlockSpec((_BN, D), lambda i: (i, 0)),
      out_shape=jax.ShapeDtypeStruct((N, D), jnp.float32),
  )(x, W, b.reshape(1, D))


def _div_body(p_ref, q_ref, c_ref, o_ref):
  o_ref[...] = (p_ref[0] + q_ref[0]) / jnp.maximum(c_ref[...], 1.0)


def _combine_div(parts, cnt):
  return pl.pallas_call(
      _div_body,
      grid=(_G,),
      in_specs=[
          pl.BlockSpec((1, _BN, D), lambda i: (0, i, 0)),
          pl.BlockSpec((1, _BN, D), lambda i: (1, i, 0)),
          pl.BlockSpec((_BN, 1), lambda i: (i, 0)),
      ],
      out_specs=pl.BlockSpec((_BN, D), lambda i: (i, 0)),
      out_shape=jax.ShapeDtypeStruct((M, D), jnp.float32),
  )(parts, parts, cnt)


def _resmm_body(h_ref, p_ref, q_ref, w_ref, b_ref, o_ref):
  a = jnp.maximum(h_ref[...] + p_ref[0] + q_ref[0], 0.0)
  o_ref[...] = jnp.dot(a, w_ref[...],
                       preferred_element_type=jnp.float32) + b_ref[...]


def _residual_relu_matmul(h, parts, W, b):
  return pl.pallas_call(
      _resmm_body,
      grid=(_G,),
      in_specs=[
          pl.BlockSpec((_BN, D), lambda i: (i, 0)),
          pl.BlockSpec((1, _BN, D), lambda i: (0, i, 0)),
          pl.BlockSpec((1, _BN, D), lambda i: (1, i, 0)),
          pl.BlockSpec((D, D), lambda i: (0, 0)),
          pl.BlockSpec((1, D), lambda i: (0, 0)),
      ],
      out_specs=pl.BlockSpec((_BN, D), lambda i: (i, 0)),
      out_shape=jax.ShapeDtypeStruct((N, D), jnp.float32),
  )(h, parts, parts, W, b.reshape(1, D))


def _resrelu_body(h_ref, p_ref, q_ref, o_ref):
  o_ref[...] = jnp.maximum(h_ref[...] + p_ref[0] + q_ref[0], 0.0)


def _residual_relu(h, parts):
  return pl.pallas_call(
      _resrelu_body,
      grid=(_G,),
      in_specs=[
          pl.BlockSpec((_BN, D), lambda i: (i, 0)),
          pl.BlockSpec((1, _BN, D), lambda i: (0, i, 0)),
          pl.BlockSpec((1, _BN, D), lambda i: (1, i, 0)),
      ],
      out_specs=pl.BlockSpec((_BN, D), lambda i: (i, 0)),
      out_shape=jax.ShapeDtypeStruct((N, D), jnp.float32),
  )(h, parts, parts)


@jax.jit
def kernel(x, hg, W1, b1, W2, b2):
  v1d = hg[0]
  e1d = hg[1]
  zd = jnp.zeros((SUB_SPAN + TAIL, D), jnp.float32)

  hist = _counts(e1d, zd)
  cnt = hist.reshape(NW, HR * 128).sum(axis=0)[:M].reshape(M, 1)

  h1 = _matmul(x, W1, b1)
  ep = _seg_sum(h1, v1d, e1d, zd)
  y1 = _combine_div(ep.reshape(NC, M, D), cnt)
  vp = _seg_sum(y1, e1d, v1d, zd)
  h2 = _residual_relu_matmul(h1, vp.reshape(NC, N, D), W2, b2)

  ep2 = _seg_sum(h2, v1d, e1d, zd)
  y2 = _combine_div(ep2.reshape(NC, M, D), cnt)
  vp2 = _seg_sum(y2, e1d, v1d, zd)
  return _residual_relu(h2, vp2.reshape(NC, N, D))


# speedup vs baseline: 10.7505x; 1.0080x over previous
import jax
import jax.numpy as jnp
from jax import lax
from jax.experimental import pallas as pl
from jax.experimental.pallas import tpu as pltpu
from jax.experimental.pallas import tpu_sc as plsc

N = 10000
M = 10000
E = 320000
D = 128

CHUNK = 160
NCHUNKS = E // CHUNK
NC, NS = 2, 16
NW = NC * NS
WCHUNKS = NCHUNKS // NW
LEFT = NCHUNKS - WCHUNKS * NW
SUB_SPAN = 624
TAIL = M - SUB_SPAN * NS
EPW = E // NW
HR = 80

_sc_mesh = plsc.VectorSubcoreMesh(core_axis_name="c", subcore_axis_name="s")


def _make_seg_sum():
  scratch = (
      [pltpu.VMEM((CHUNK,), jnp.int32) for _ in range(8)] +
      [pltpu.VMEM((CHUNK, D), jnp.float32) for _ in range(2)] +
      [pltpu.VMEM_SHARED((M, D), jnp.float32)] +
      [pltpu.SemaphoreType.DMA for _ in range(8)]
  )

  def body(src_hbm, gidx_hbm, sidx_hbm, zd_hbm, out_hbm, *refs):
    gb = refs[0:4]
    sb = refs[4:8]
    rows = refs[8:10]
    acc = refs[10]
    mg = refs[11:13]
    ms = refs[13:15]
    mi = refs[15:19]

    cid = lax.axis_index("c")
    sid = lax.axis_index("s")
    wid = cid * NS + sid
    acc_base = sid * SUB_SPAN
    off_w = wid * WCHUNKS

    def idx_start(j, p):
      base = (off_w + j) * CHUNK
      pltpu.async_copy(gidx_hbm.at[pl.ds(base, CHUNK)], gb[p], mi[p])
      pltpu.async_copy(sidx_hbm.at[pl.ds(base, CHUNK)], sb[p], mi[p])

    def idx_wait(j, p):
      base = (off_w + j) * CHUNK
      pltpu.make_async_copy(gidx_hbm.at[pl.ds(base, CHUNK)], gb[p],
                            mi[p]).wait()
      pltpu.make_async_copy(sidx_hbm.at[pl.ds(base, CHUNK)], sb[p],
                            mi[p]).wait()

    def gather(p, b):
      pltpu.async_copy(src_hbm.at[gb[p]], rows[b], mg[b])

    def gather_wait(p, b):
      pltpu.make_async_copy(src_hbm.at[gb[p]], rows[b], mg[b]).wait()

    def scatter(p, b):
      pltpu.async_copy(rows[b], acc.at[sb[p]], ms[b], add=True)

    def scatter_wait(p, b):
      pltpu.make_async_copy(rows[b], acc.at[sb[p]], ms[b]).wait()

    idx_start(0, 0)
    idx_start(1, 1)
    idx_start(2, 2)

    pltpu.sync_copy(zd_hbm.at[pl.ds(0, SUB_SPAN)],
                    acc.at[pl.ds(acc_base, SUB_SPAN)])

    @pl.when(sid == NS - 1)
    def _():
      pltpu.sync_copy(zd_hbm.at[pl.ds(0, TAIL)],
                      acc.at[pl.ds(NS * SUB_SPAN, TAIL)])

    idx_wait(0, 0)
    gather(0, 0)
    plsc.subcore_barrier()

    @pl.loop(0, WCHUNKS // 4)
    def _(k):
      for t in range(4):
        j = 4 * k + t
        b = t % 2
        gather_wait(t, b)
        if t == 0:
          @pl.when(k > 0)
          def _():
            scatter_wait(3, 1 - b)
        else:
          scatter_wait(t - 1, 1 - b)
        if t == 3:
          @pl.when(k < WCHUNKS // 4 - 1)
          def _():
            idx_start(j + 3, (t + 3) % 4)
        else:
          idx_start(j + 3, (t + 3) % 4)
        idx_wait(j + 1, (t + 1) % 4)
        gather((t + 1) % 4, 1 - b)
        scatter(t, b)

    j = WCHUNKS - 2
    gather_wait(0, 0)
    scatter_wait(3, 1)
    idx_wait(j + 1, 1)
    gather(1, 1)
    scatter(0, 0)
    gather_wait(1, 1)
    scatter_wait(0, 0)
    scatter(1, 1)
    scatter_wait(1, 1)

    @pl.when(wid < LEFT)
    def _():
      base = (NW * WCHUNKS + wid) * CHUNK
      pltpu.sync_copy(gidx_hbm.at[pl.ds(base, CHUNK)], gb[0])
      pltpu.sync_copy(sidx_hbm.at[pl.ds(base, CHUNK)], sb[0])
      pltpu.async_copy(src_hbm.at[gb[0]], rows[0], mg[0]).wait()
      pltpu.sync_copy(rows[0], acc.at[sb[0]], add=True)

    plsc.subcore_barrier()

    out_base = cid * M + acc_base
    pltpu.sync_copy(acc.at[pl.ds(acc_base, SUB_SPAN)],
                    out_hbm.at[pl.ds(out_base, SUB_SPAN)])

    @pl.when(sid == NS - 1)
    def _():
      pltpu.sync_copy(acc.at[pl.ds(NS * SUB_SPAN, TAIL)],
                      out_hbm.at[pl.ds(cid * M + NS * SUB_SPAN, TAIL)])

  return pl.kernel(body, out_type=jax.ShapeDtypeStruct((NC * M, D), jnp.float32),
                   mesh=_sc_mesh, scratch_types=scratch, name="seg_sum")


def _make_counts():
  scratch = [
      pltpu.VMEM((EPW,), jnp.int32),
      pltpu.VMEM((HR, 128), jnp.float32),
  ]

  def body(sidx_hbm, zd_hbm, out_hbm, sbuf, hist):
    cid = lax.axis_index("c")
    sid = lax.axis_index("s")
    wid = cid * NS + sid

    pltpu.sync_copy(zd_hbm.at[pl.ds(0, HR)], hist)
    pltpu.sync_copy(sidx_hbm.at[pl.ds(wid * EPW, EPW)], sbuf)

    ones = jnp.ones((16,), jnp.float32)

    @pl.loop(0, EPW // 16)
    def _(i):
      idx = sbuf[pl.ds(i * 16, 16)]
      hi = lax.shift_right_logical(idx, 7)
      lo = lax.bitwise_and(idx, 127)
      plsc.addupdate_scatter(hist, [hi, lo], ones)

    pltpu.sync_copy(hist, out_hbm.at[pl.ds(wid * HR, HR)])

  return pl.kernel(
      body, out_type=jax.ShapeDtypeStruct((NW * HR, 128), jnp.float32),
      mesh=_sc_mesh, scratch_types=scratch, name="edge_counts",
      compiler_params=pltpu.CompilerParams(needs_layout_passes=False))


_seg_sum = _make_seg_sum()
_counts = _make_counts()


_BN = 1000
_G = N // _BN


def _mm_body(x_ref, w_ref, b_ref, o_ref):
  o_ref[...] = jnp.dot(x_ref[...], w_ref[...],
                       preferred_element_type=jnp.float32) + b_ref[...]


def _matmul(x, W, b):
  return pl.pallas_call(
      _mm_body,
      grid=(_G,),
      in_specs=[
          pl.BlockSpec((_BN, D), lambda i: (i, 0)),
          pl.BlockSpec((D, D), lambda i: (0, 0)),
          pl.BlockSpec((1, D), lambda i: (0, 0)),
      ],
      out_specs=pl.BlockSpec((_BN, D), lambda i: (i, 0)),
      out_shape=jax.ShapeDtypeStruct((N, D), jnp.float32),
  )(x, W, b.reshape(1, D))


def _div_body(p_ref, q_ref, c_ref, o_ref):
  o_ref[...] = (p_ref[0] + q_ref[0]) / jnp.maximum(c_ref[...], 1.0)


def _combine_div(parts, cnt):
  return pl.pallas_call(
      _div_body,
      grid=(_G,),
      in_specs=[
          pl.BlockSpec((1, _BN, D), lambda i: (0, i, 0)),
          pl.BlockSpec((1, _BN, D), lambda i: (1, i, 0)),
          pl.BlockSpec((_BN, 1), lambda i: (i, 0)),
      ],
      out_specs=pl.BlockSpec((_BN, D), lambda i: (i, 0)),
      out_shape=jax.ShapeDtypeStruct((M, D), jnp.float32),
  )(parts, parts, cnt)


def _resmm_body(h_ref, p_ref, q_ref, w_ref, b_ref, o_ref):
  a = jnp.maximum(h_ref[...] + p_ref[0] + q_ref[0], 0.0)
  o_ref[...] = jnp.dot(a, w_ref[...],
                       preferred_element_type=jnp.float32) + b_ref[...]


def _residual_relu_matmul(h, parts, W, b):
  return pl.pallas_call(
      _resmm_body,
      grid=(_G,),
      in_specs=[
          pl.BlockSpec((_BN, D), lambda i: (i, 0)),
          pl.BlockSpec((1, _BN, D), lambda i: (0, i, 0)),
          pl.BlockSpec((1, _BN, D), lambda i: (1, i, 0)),
          pl.BlockSpec((D, D), lambda i: (0, 0)),
          pl.BlockSpec((1, D), lambda i: (0, 0)),
      ],
      out_specs=pl.BlockSpec((_BN, D), lambda i: (i, 0)),
      out_shape=jax.ShapeDtypeStruct((N, D), jnp.float32),
  )(h, parts, parts, W, b.reshape(1, D))


def _resrelu_body(h_ref, p_ref, q_ref, o_ref):
  o_ref[...] = jnp.maximum(h_ref[...] + p_ref[0] + q_ref[0], 0.0)


def _residual_relu(h, parts):
  return pl.pallas_call(
      _resrelu_body,
      grid=(_G,),
      in_specs=[
          pl.BlockSpec((_BN, D), lambda i: (i, 0)),
          pl.BlockSpec((1, _BN, D), lambda i: (0, i, 0)),
          pl.BlockSpec((1, _BN, D), lambda i: (1, i, 0)),
      ],
      out_specs=pl.BlockSpec((_BN, D), lambda i: (i, 0)),
      out_shape=jax.ShapeDtypeStruct((N, D), jnp.float32),
  )(h, parts, parts)


@jax.jit
def kernel(x, hg, W1, b1, W2, b2):
  v1d = hg[0]
  e1d = hg[1]
  zd = jnp.zeros((SUB_SPAN + TAIL, D), jnp.float32)

  hist = _counts(e1d, zd)
  cnt = hist.reshape(NW, HR * 128).sum(axis=0)[:M].reshape(M, 1)

  h1 = _matmul(x, W1, b1)
  ep = _seg_sum(h1, v1d, e1d, zd)
  y1 = _combine_div(ep.reshape(NC, M, D), cnt)
  vp = _seg_sum(y1, e1d, v1d, zd)
  h2 = _residual_relu_matmul(h1, vp.reshape(NC, N, D), W2, b2)

  ep2 = _seg_sum(h2, v1d, e1d, zd)
  y2 = _combine_div(ep2.reshape(NC, M, D), cnt)
  vp2 = _seg_sum(y2, e1d, v1d, zd)
  return _residual_relu(h2, vp2.reshape(NC, N, D))
